# baseline (speedup 1.0000x reference)
import jax
import jax.numpy as jnp
from jax import lax
from jax.experimental import pallas as pl
from jax.experimental.pallas import tpu as pltpu
from jax.experimental.pallas import tpu_sc as plsc

N_NODES_ = 10000
N_EDGES_ = 320000
D_ = 128

_info = plsc.get_sparse_core_info()
NC = _info.num_cores
NS = _info.num_subcores
NW = NC * NS

E_PER_W = N_EDGES_ // NW
CHUNK = 200
N_CHUNKS = E_PER_W // CHUNK
NBUF = 4


def _gather_body(idx_hbm, x_hbm, out_hbm, idx_all, rows_v, sem_g, sem_o):
    wid = lax.axis_index("s") * NC + lax.axis_index("c")
    base = wid * E_PER_W

    pltpu.sync_copy(idx_hbm.at[pl.ds(base, E_PER_W)], idx_all)

    def gather(g, b):
        return pltpu.make_async_copy(
            x_hbm.at[idx_all.at[pl.ds(g * CHUNK, CHUNK)]], rows_v[b], sem_g[b])

    def store(g, b):
        return pltpu.make_async_copy(
            rows_v[b], out_hbm.at[pl.ds(base + g * CHUNK, CHUNK)], sem_o[b])

    def step(g, b, i_pos):
        pb = (b - 1) % NBUF

        if i_pos is None:
            store(g, b).wait()
            gather(g, b).start()
            gather(g - 1, pb).wait()
            store(g - 1, pb).start()
            return

        @pl.when(i_pos)
        def _():
            store(g, b).wait()

        gather(g, b).start()

        @pl.when(g > 0)
        def _():
            gather(g - 1, pb).wait()
            store(g - 1, pb).start()

    def quad(i, carry):
        for b in range(NBUF):
            step(i * NBUF + b, b, i > 0)
        return carry

    nquad = N_CHUNKS // NBUF
    lax.fori_loop(0, nquad, quad, 0)
    for g in range(nquad * NBUF, N_CHUNKS):
        step(g, g % NBUF, None)

    last = N_CHUNKS - 1
    gather(last, last % NBUF).wait()
    store(last, last % NBUF).start()
    for b in range(NBUF):
        store(last - ((last - b) % NBUF), b).wait()


@jax.jit
def kernel(x, edge_index):
    idx = edge_index[1]
    mesh = plsc.VectorSubcoreMesh(core_axis_name="c", subcore_axis_name="s")
    run = pl.kernel(
        _gather_body,
        out_type=jax.ShapeDtypeStruct((N_EDGES_, D_), jnp.float32),
        mesh=mesh,
        scratch_types=[
            pltpu.VMEM((E_PER_W,), jnp.int32),
            tuple(pltpu.VMEM((CHUNK, D_), jnp.float32) for _ in range(NBUF)),
            tuple(pltpu.SemaphoreType.DMA for _ in range(NBUF)),
            tuple(pltpu.SemaphoreType.DMA for _ in range(NBUF)),
        ],
    )
    return run(idx, x)


# speedup vs baseline: 1.4777x; 1.4777x over previous
import jax
import jax.numpy as jnp
from jax import lax
from jax.experimental import pallas as pl
from jax.experimental.pallas import tpu as pltpu
from jax.experimental.pallas import tpu_sc as plsc

N_NODES_ = 10000
N_EDGES_ = 320000
D_ = 128

_info = plsc.get_sparse_core_info()
NC = _info.num_cores
NS = _info.num_subcores
NW = NC * NS

E_PER_W = N_EDGES_ // NW
CHUNK = 80
N_CHUNKS = E_PER_W // CHUNK
NBUF = 4
ROWS_PER_TILE = 624
ROWS_REMAINDER = N_NODES_ - NS * ROWS_PER_TILE


def _gather_body(idx_hbm, x_hbm, out_hbm, idx_all, table_s, rows_v,
                 sem_g, sem_o):
    sid = lax.axis_index("s")
    wid = sid * NC + lax.axis_index("c")
    base = wid * E_PER_W

    pltpu.sync_copy(x_hbm.at[pl.ds(sid * ROWS_PER_TILE, ROWS_PER_TILE)],
                    table_s.at[pl.ds(sid * ROWS_PER_TILE, ROWS_PER_TILE)])

    @pl.when(sid == 0)
    def _():
        tail = NS * ROWS_PER_TILE
        pltpu.sync_copy(x_hbm.at[pl.ds(tail, ROWS_REMAINDER)],
                        table_s.at[pl.ds(tail, ROWS_REMAINDER)])

    pltpu.sync_copy(idx_hbm.at[pl.ds(base, E_PER_W)], idx_all)
    plsc.subcore_barrier()

    def gather(g, b):
        return pltpu.make_async_copy(
            table_s.at[idx_all.at[pl.ds(g * CHUNK, CHUNK)]],
            rows_v[b], sem_g[b])

    def store(g, b):
        return pltpu.make_async_copy(
            rows_v[b], out_hbm.at[pl.ds(base + g * CHUNK, CHUNK)], sem_o[b])

    def step(g, b, i_pos):
        pb = (b - 1) % NBUF

        if i_pos is None:
            store(g, b).wait()
            gather(g, b).start()
            gather(g - 1, pb).wait()
            store(g - 1, pb).start()
            return

        @pl.when(i_pos)
        def _():
            store(g, b).wait()

        gather(g, b).start()

        @pl.when(g > 0)
        def _():
            gather(g - 1, pb).wait()
            store(g - 1, pb).start()

    def quad(i, carry):
        for b in range(NBUF):
            step(i * NBUF + b, b, i > 0)
        return carry

    nquad = N_CHUNKS // NBUF
    lax.fori_loop(0, nquad, quad, 0)
    for g in range(nquad * NBUF, N_CHUNKS):
        step(g, g % NBUF, None)

    last = N_CHUNKS - 1
    gather(last, last % NBUF).wait()
    store(last, last % NBUF).start()
    for b in range(NBUF):
        store(last - ((last - b) % NBUF), b).wait()


@jax.jit
def kernel(x, edge_index):
    idx = edge_index[1]
    mesh = plsc.VectorSubcoreMesh(core_axis_name="c", subcore_axis_name="s")
    run = pl.kernel(
        _gather_body,
        out_type=jax.ShapeDtypeStruct((N_EDGES_, D_), jnp.float32),
        mesh=mesh,
        scratch_types=[
            pltpu.VMEM((E_PER_W,), jnp.int32),
            pltpu.VMEM_SHARED((N_NODES_, D_), jnp.float32),
            tuple(pltpu.VMEM((CHUNK, D_), jnp.float32) for _ in range(NBUF)),
            tuple(pltpu.SemaphoreType.DMA for _ in range(NBUF)),
            tuple(pltpu.SemaphoreType.DMA for _ in range(NBUF)),
        ],
    )
    return run(idx, x)


# speedup vs baseline: 1.4906x; 1.0087x over previous
import jax
import jax.numpy as jnp
from jax import lax
from jax.experimental import pallas as pl
from jax.experimental.pallas import tpu as pltpu
from jax.experimental.pallas import tpu_sc as plsc

N_NODES_ = 10000
N_EDGES_ = 320000
D_ = 128

_info = plsc.get_sparse_core_info()
NC = _info.num_cores
NS = _info.num_subcores
NW = NC * NS

E_PER_W = N_EDGES_ // NW
CHUNK = 80
N_CHUNKS = E_PER_W // CHUNK
NBUF = 4
ROWS_PER_TILE = 624
ROWS_REMAINDER = N_NODES_ - NS * ROWS_PER_TILE


def _gather_body(idx_hbm, x_hbm, out_hbm, idx_all, table_s, rows_v,
                 sem_g, sem_o, sem_s):
    sid = lax.axis_index("s")
    wid = sid * NC + lax.axis_index("c")
    base = wid * E_PER_W

    stage_t = pltpu.make_async_copy(
        x_hbm.at[pl.ds(sid * ROWS_PER_TILE, ROWS_PER_TILE)],
        table_s.at[pl.ds(sid * ROWS_PER_TILE, ROWS_PER_TILE)], sem_s[0])
    stage_i = pltpu.make_async_copy(
        idx_hbm.at[pl.ds(base, E_PER_W)], idx_all, sem_s[1])
    stage_t.start()
    stage_i.start()

    @pl.when(sid == 0)
    def _():
        tail = NS * ROWS_PER_TILE
        pltpu.sync_copy(x_hbm.at[pl.ds(tail, ROWS_REMAINDER)],
                        table_s.at[pl.ds(tail, ROWS_REMAINDER)])

    def gather_from(src, g, b):
        return pltpu.make_async_copy(
            src.at[idx_all.at[pl.ds(g * CHUNK, CHUNK)]],
            rows_v[b], sem_g[b])

    def gather(g, b):
        return gather_from(table_s, g, b)

    def store(g, b):
        return pltpu.make_async_copy(
            rows_v[b], out_hbm.at[pl.ds(base + g * CHUNK, CHUNK)], sem_o[b])

    stage_i.wait()
    for b in range(NBUF):
        gather_from(x_hbm, b, b).start()
    for b in range(NBUF - 1):
        gather(b, b).wait()
        store(b, b).start()
    stage_t.wait()
    plsc.subcore_barrier()

    def step(g, b, i_pos):
        pb = (b - 1) % NBUF

        if i_pos is None:
            store(g, b).wait()
            gather(g, b).start()
            gather(g - 1, pb).wait()
            store(g - 1, pb).start()
            return

        @pl.when(i_pos)
        def _():
            store(g, b).wait()

        gather(g, b).start()

        @pl.when(g > 0)
        def _():
            gather(g - 1, pb).wait()
            store(g - 1, pb).start()

    def quad(i, carry):
        for b in range(NBUF):
            step(i * NBUF + b, b, i > 0)
        return carry

    nquad = N_CHUNKS // NBUF
    lax.fori_loop(1, nquad, quad, 0)
    for g in range(nquad * NBUF, N_CHUNKS):
        step(g, g % NBUF, None)

    last = N_CHUNKS - 1
    gather(last, last % NBUF).wait()
    store(last, last % NBUF).start()
    for b in range(NBUF):
        store(last - ((last - b) % NBUF), b).wait()


@jax.jit
def kernel(x, edge_index):
    idx = edge_index[1]
    mesh = plsc.VectorSubcoreMesh(core_axis_name="c", subcore_axis_name="s")
    run = pl.kernel(
        _gather_body,
        out_type=jax.ShapeDtypeStruct((N_EDGES_, D_), jnp.float32),
        mesh=mesh,
        scratch_types=[
            pltpu.VMEM((E_PER_W,), jnp.int32),
            pltpu.VMEM_SHARED((N_NODES_, D_), jnp.float32),
            tuple(pltpu.VMEM((CHUNK, D_), jnp.float32) for _ in range(NBUF)),
            tuple(pltpu.SemaphoreType.DMA for _ in range(NBUF)),
            tuple(pltpu.SemaphoreType.DMA for _ in range(NBUF)),
            tuple(pltpu.SemaphoreType.DMA for _ in range(2)),
        ],
    )
    return run(idx, x)


# speedup vs baseline: 1.5045x; 1.0093x over previous
import jax
import jax.numpy as jnp
from jax import lax
from jax.experimental import pallas as pl
from jax.experimental.pallas import tpu as pltpu
from jax.experimental.pallas import tpu_sc as plsc

N_NODES_ = 10000
N_EDGES_ = 320000
D_ = 128

_info = plsc.get_sparse_core_info()
NC = _info.num_cores
NS = _info.num_subcores
NW = NC * NS

E_PER_W = N_EDGES_ // NW
CHUNK = 80
N_CHUNKS = E_PER_W // CHUNK
SUPER = 2 * CHUNK
N_SUPER = N_CHUNKS // 2
ROWS_PER_TILE = 624
ROWS_REMAINDER = N_NODES_ - NS * ROWS_PER_TILE


def _gather_body(idx_hbm, x_hbm, out_hbm, idx_all, table_s, rows_v,
                 sem_g, sem_o, sem_s):
    sid = lax.axis_index("s")
    wid = sid * NC + lax.axis_index("c")
    base = wid * E_PER_W

    stage_t = pltpu.make_async_copy(
        x_hbm.at[pl.ds(sid * ROWS_PER_TILE, ROWS_PER_TILE)],
        table_s.at[pl.ds(sid * ROWS_PER_TILE, ROWS_PER_TILE)], sem_s[0])
    stage_i = pltpu.make_async_copy(
        idx_hbm.at[pl.ds(base, E_PER_W)], idx_all, sem_s[1])
    stage_t.start()
    stage_i.start()

    @pl.when(sid == 0)
    def _():
        tail = NS * ROWS_PER_TILE
        pltpu.sync_copy(x_hbm.at[pl.ds(tail, ROWS_REMAINDER)],
                        table_s.at[pl.ds(tail, ROWS_REMAINDER)])

    stage_t.wait()
    stage_i.wait()
    plsc.subcore_barrier()

    def gather(g, s, sub):
        return pltpu.make_async_copy(
            table_s.at[idx_all.at[pl.ds(g * CHUNK, CHUNK)]],
            rows_v[s].at[pl.ds(sub * CHUNK, CHUNK)], sem_g[2 * s + sub])

    def store_super(S, s):
        return pltpu.make_async_copy(
            rows_v[s], out_hbm.at[pl.ds(base + S * SUPER, SUPER)], sem_o[s])

    def step(S, s, S_pos2):
        ps = 1 - s

        @pl.when(S_pos2)
        def _():
            store_super(S, s).wait()

        gather(2 * S, s, 0).start()
        gather(2 * S + 1, s, 1).start()

        @pl.when(S > 0)
        def _():
            gather(2 * S - 2, ps, 0).wait()
            gather(2 * S - 1, ps, 1).wait()
            store_super(S - 1, ps).start()

    def pair(i, carry):
        step(2 * i, 0, i > 0)
        step(2 * i + 1, 1, i > 0)
        return carry

    lax.fori_loop(0, N_SUPER // 2, pair, 0)

    lastS = N_SUPER - 1
    gather(2 * lastS, 1, 0).wait()
    gather(2 * lastS + 1, 1, 1).wait()
    store_super(lastS, 1).start()

    g = N_CHUNKS - 1
    store_super(lastS - 1, 0).wait()
    gather(g, 0, 0).start()
    gather(g, 0, 0).wait()
    tail_store = pltpu.make_async_copy(
        rows_v[0].at[pl.ds(0, CHUNK)],
        out_hbm.at[pl.ds(base + g * CHUNK, CHUNK)], sem_o[0])
    tail_store.start()
    store_super(lastS, 1).wait()
    tail_store.wait()


@jax.jit
def kernel(x, edge_index):
    idx = edge_index[1]
    mesh = plsc.VectorSubcoreMesh(core_axis_name="c", subcore_axis_name="s")
    run = pl.kernel(
        _gather_body,
        out_type=jax.ShapeDtypeStruct((N_EDGES_, D_), jnp.float32),
        mesh=mesh,
        scratch_types=[
            pltpu.VMEM((E_PER_W,), jnp.int32),
            pltpu.VMEM_SHARED((N_NODES_, D_), jnp.float32),
            tuple(pltpu.VMEM((SUPER, D_), jnp.float32) for _ in range(2)),
            tuple(pltpu.SemaphoreType.DMA for _ in range(4)),
            tuple(pltpu.SemaphoreType.DMA for _ in range(2)),
            tuple(pltpu.SemaphoreType.DMA for _ in range(2)),
        ],
    )
    return run(idx, x)
